# baseline (speedup 1.0000x reference)
import functools

import jax
import jax.numpy as jnp
import numpy as np
from jax import lax
from jax.experimental import pallas as pl
from jax.experimental.pallas import tpu as pltpu
from jax.experimental.pallas import tpu_sc as plsc

_RATIO = 0.75

_NC = 2
_NS = 16
_NW = _NC * _NS


def _f_idx_jnp(T: int, B: int):
    base = jax.random.key(42)
    cols = [jax.random.permutation(jax.random.fold_in(base, j), T) for j in range(B)]
    return jnp.stack(cols, axis=-1)


@functools.lru_cache(maxsize=None)
def _host_indices(T: int, B: int):
    try:
        cpu = jax.devices("cpu")[0]
        with jax.default_device(cpu), jax.ensure_compile_time_eval():
            f_idx = _f_idx_jnp(T, B)
        return np.asarray(jax.device_get(f_idx))
    except Exception:
        return None


@functools.lru_cache(maxsize=None)
def _make_gather(rows: int, C: int, K: int, nidx: int):
    nchunk_per_w = rows // (_NW * K)
    rpw = rows // _NW
    ipw = nidx // _NW
    nbuf = 5

    @functools.partial(
        pl.kernel,
        mesh=plsc.VectorSubcoreMesh(core_axis_name="c", subcore_axis_name="s"),
        out_type=(
            jax.ShapeDtypeStruct((rows, C), jnp.float32),
            jax.ShapeDtypeStruct((nidx,), jnp.int32),
            jax.ShapeDtypeStruct((nidx,), jnp.int32),
        ),
        scratch_types=[
            pltpu.VMEM((nchunk_per_w, K), jnp.int32),
        ]
        + [pltpu.VMEM((K, C), jnp.float32)] * nbuf
        + [pltpu.SemaphoreType.DMA] * nbuf,
    )
    def gather_kernel(table, idxs, fi, out, f_out, b_out, idx_v, *rest):
        bufs = rest[:nbuf]
        sems = rest[nbuf:]
        wid = lax.axis_index("s") * _NC + lax.axis_index("c")
        base = wid * rpw
        pltpu.sync_copy(idxs.at[wid], idx_v)
        cps = [None] * nbuf
        for j in range(min(nbuf - 1, nchunk_per_w)):
            cps[j] = pltpu.async_copy(table.at[idx_v.at[j]], bufs[j], sems[j])
        ib = wid * ipw
        pltpu.sync_copy(fi.at[pl.ds(ib, ipw)], f_out.at[pl.ds(ib, ipw)])
        pltpu.sync_copy(fi.at[pl.ds(ib, ipw)], b_out.at[pl.ds(ib, ipw)])
        for j in range(nchunk_per_w):
            nj = j + nbuf - 1
            if nj < nchunk_per_w:
                p = nj % nbuf
                cps[p] = pltpu.async_copy(table.at[idx_v.at[nj]], bufs[p], sems[p])
            cps[j % nbuf].wait()
            pltpu.sync_copy(bufs[j % nbuf], out.at[pl.ds(base + j * K, K)])

    return gather_kernel


def kernel(pths):
    T, B, C = pths.shape
    keep = int(T * (1 - _RATIO))
    rows = keep * B
    K = 32

    fi = _host_indices(T, B)
    if fi is not None:
        flat = (fi[:keep].astype(np.int64) * B + np.arange(B)[None, :]).astype(
            np.int32
        )
        idxs = jnp.asarray(flat.reshape(_NW, rows // (_NW * K), K))
        fi_arr = jnp.asarray(fi.reshape(-1))
    else:
        f_idx_t = _f_idx_jnp(T, B)
        flat = f_idx_t[:keep] * B + jnp.arange(B, dtype=jnp.int32)[None, :]
        idxs = flat.reshape(_NW, rows // (_NW * K), K)
        fi_arr = f_idx_t.reshape(-1)

    table = pths.reshape(T * B, C)
    out, f_flat, b_flat = _make_gather(rows, C, K, T * B)(table, idxs, fi_arr)
    shuffled = out.reshape(keep, B, C)
    return (shuffled, f_flat.reshape(T, B), b_flat.reshape(T, B))


# speedup vs baseline: 1.1448x; 1.1448x over previous
import functools

import jax
import jax.numpy as jnp
import numpy as np
from jax import lax
from jax.experimental import pallas as pl
from jax.experimental.pallas import tpu as pltpu
from jax.experimental.pallas import tpu_sc as plsc

_RATIO = 0.75

_NC = 2
_NS = 16
_NW = _NC * _NS


def _f_idx_jnp(T: int, B: int):
    base = jax.random.key(42)
    cols = [jax.random.permutation(jax.random.fold_in(base, j), T) for j in range(B)]
    return jnp.stack(cols, axis=-1)


@functools.lru_cache(maxsize=None)
def _host_indices(T: int, B: int):
    try:
        cpu = jax.devices("cpu")[0]
        with jax.default_device(cpu), jax.ensure_compile_time_eval():
            f_idx = _f_idx_jnp(T, B)
        return np.asarray(jax.device_get(f_idx))
    except Exception:
        return None


@functools.lru_cache(maxsize=None)
def _make_gather(keep: int, B: int, C: int, T: int, K: int):
    rows = keep * B
    nchunk_per_w = rows // (_NW * K)
    rpw = rows // _NW
    tpw = T // _NW
    per_t = B // K
    nbuf = 5

    @functools.partial(
        pl.kernel,
        mesh=plsc.VectorSubcoreMesh(core_axis_name="c", subcore_axis_name="s"),
        out_type=(
            jax.ShapeDtypeStruct((keep, B, C), jnp.float32),
            jax.ShapeDtypeStruct((T, B), jnp.int32),
            jax.ShapeDtypeStruct((T, B), jnp.int32),
        ),
        scratch_types=[
            pltpu.VMEM((nchunk_per_w, K), jnp.int32),
            pltpu.SemaphoreType.DMA,
            pltpu.SemaphoreType.DMA,
        ]
        + [pltpu.VMEM((K, C), jnp.float32)] * nbuf
        + [pltpu.SemaphoreType.DMA] * nbuf,
    )
    def gather_kernel(table, idxs, fi, out, f_out, b_out, idx_v, fsem, bsem, *rest):
        bufs = rest[:nbuf]
        sems = rest[nbuf:]
        wid = lax.axis_index("s") * _NC + lax.axis_index("c")
        pltpu.sync_copy(idxs.at[wid], idx_v)
        cps = [None] * nbuf
        for j in range(min(nbuf - 1, nchunk_per_w)):
            cps[j] = pltpu.async_copy(table.at[idx_v.at[j]], bufs[j], sems[j])
        ib = wid * tpw
        fcp = pltpu.async_copy(
            fi.at[pl.ds(ib, tpw)], f_out.at[pl.ds(ib, tpw)], fsem
        )
        bcp = pltpu.async_copy(
            fi.at[pl.ds(ib, tpw)], b_out.at[pl.ds(ib, tpw)], bsem
        )
        t0 = wid * (rpw // B)
        for j in range(nchunk_per_w):
            nj = j + nbuf - 1
            if nj < nchunk_per_w:
                p = nj % nbuf
                cps[p] = pltpu.async_copy(table.at[idx_v.at[nj]], bufs[p], sems[p])
            cps[j % nbuf].wait()
            pltpu.sync_copy(
                bufs[j % nbuf],
                out.at[t0 + j // per_t, pl.ds((j % per_t) * K, K)],
            )
        fcp.wait()
        bcp.wait()

    return gather_kernel


def kernel(pths):
    T, B, C = pths.shape
    keep = int(T * (1 - _RATIO))
    rows = keep * B
    K = 32

    fi = _host_indices(T, B)
    if fi is not None:
        flat = (fi[:keep].astype(np.int64) * B + np.arange(B)[None, :]).astype(
            np.int32
        )
        idxs = jnp.asarray(flat.reshape(_NW, rows // (_NW * K), K))
        fi_arr = jnp.asarray(fi)
    else:
        fi_arr = _f_idx_jnp(T, B)
        flat = fi_arr[:keep] * B + jnp.arange(B, dtype=jnp.int32)[None, :]
        idxs = flat.reshape(_NW, rows // (_NW * K), K)

    table = pths.reshape(T * B, C)
    shuffled, f_idx, b_idx = _make_gather(keep, B, C, T, K)(table, idxs, fi_arr)
    return (shuffled, f_idx, b_idx)


# speedup vs baseline: 1.2119x; 1.0587x over previous
import functools

import jax
import jax.numpy as jnp
import numpy as np
from jax import lax
from jax.experimental import pallas as pl
from jax.experimental.pallas import tpu as pltpu
from jax.experimental.pallas import tpu_sc as plsc

_RATIO = 0.75

_NC = 2
_NS = 16
_NW = _NC * _NS


def _f_idx_jnp(T: int, B: int):
    base = jax.random.key(42)
    cols = [jax.random.permutation(jax.random.fold_in(base, j), T) for j in range(B)]
    return jnp.stack(cols, axis=-1)


@functools.lru_cache(maxsize=None)
def _host_indices(T: int, B: int):
    try:
        cpu = jax.devices("cpu")[0]
        with jax.default_device(cpu), jax.ensure_compile_time_eval():
            f_idx = _f_idx_jnp(T, B)
        return np.asarray(jax.device_get(f_idx))
    except Exception:
        return None


@functools.lru_cache(maxsize=None)
def _make_gather(keep: int, B: int, C: int, T: int, K: int):
    rows = keep * B
    nchunk_per_w = rows // (_NW * K)
    rpw = rows // _NW
    tpw = T // _NW
    per_t = B // K
    nbuf = 5

    @functools.partial(
        pl.kernel,
        mesh=plsc.VectorSubcoreMesh(core_axis_name="c", subcore_axis_name="s"),
        out_type=jax.ShapeDtypeStruct((keep, B, C), jnp.float32),
        scratch_types=[
            pltpu.VMEM((nchunk_per_w, K), jnp.int32),
        ]
        + [pltpu.VMEM((K, C), jnp.float32)] * nbuf
        + [pltpu.SemaphoreType.DMA] * nbuf,
    )
    def gather_kernel(table, idxs, out, idx_v, *rest):
        bufs = rest[:nbuf]
        sems = rest[nbuf:]
        wid = lax.axis_index("s") * _NC + lax.axis_index("c")
        pltpu.sync_copy(idxs.at[wid], idx_v)
        cps = [None] * nbuf
        for j in range(min(nbuf - 1, nchunk_per_w)):
            cps[j] = pltpu.async_copy(table.at[idx_v.at[j]], bufs[j], sems[j])
        t0 = wid * (rpw // B)
        for j in range(nchunk_per_w):
            nj = j + nbuf - 1
            if nj < nchunk_per_w:
                p = nj % nbuf
                cps[p] = pltpu.async_copy(table.at[idx_v.at[nj]], bufs[p], sems[p])
            cps[j % nbuf].wait()
            pltpu.sync_copy(
                bufs[j % nbuf],
                out.at[t0 + j // per_t, pl.ds((j % per_t) * K, K)],
            )

    return gather_kernel


def kernel(pths):
    T, B, C = pths.shape
    keep = int(T * (1 - _RATIO))
    rows = keep * B
    K = 32

    fi = _host_indices(T, B)
    if fi is not None:
        flat = (fi[:keep].astype(np.int64) * B + np.arange(B)[None, :]).astype(
            np.int32
        )
        idxs = jnp.asarray(flat.reshape(_NW, rows // (_NW * K), K))
        f_idx = jnp.asarray(fi)
    else:
        f_idx = _f_idx_jnp(T, B)
        flat = f_idx[:keep] * B + jnp.arange(B, dtype=jnp.int32)[None, :]
        idxs = flat.reshape(_NW, rows // (_NW * K), K)

    table = pths.reshape(T * B, C)
    shuffled = _make_gather(keep, B, C, T, K)(table, idxs)
    return (shuffled, f_idx, f_idx)
